# Initial kernel scaffold; baseline (speedup 1.0000x reference)
import functools

import jax
import jax.numpy as jnp
from jax import lax
from jax.experimental import pallas as pl
from jax.experimental.pallas import tpu as pltpu
from jax.experimental.pallas import tpu_sc as plsc

N_NODES = 10000
N_EDGES = 320000
D = 128

NC = 2
NS = 16
NW = NC * NS

CHUNK = 128
N_CHUNKS = N_EDGES // CHUNK
CHUNKS_PER_TILE = -(-N_CHUNKS // NW)
N_CHUNKS_PAD = CHUNKS_PER_TILE * NW
E_PAD = N_CHUNKS_PAD * CHUNK

ACC_ROWS = N_NODES + 16
ZERO_ROWS_PER_TILE = ACC_ROWS // NS
OUT_ROWS_PER_TILE = N_NODES // NS


@functools.partial(
    pl.kernel,
    out_type=jax.ShapeDtypeStruct((NC, N_NODES, D), jnp.float32),
    mesh=plsc.VectorSubcoreMesh(core_axis_name="c", subcore_axis_name="s"),
    scratch_types=[
        pltpu.VMEM((CHUNKS_PER_TILE, CHUNK), jnp.int32),
        pltpu.VMEM((CHUNKS_PER_TILE, CHUNK), jnp.int32),
        pltpu.VMEM((CHUNK, D), jnp.float32),
        pltpu.VMEM_SHARED((ACC_ROWS, D), jnp.float32),
        pltpu.SemaphoreType.DMA,
    ],
)
def _sc_segment_sum(x_hbm, src_hbm, dst_hbm, zero_hbm, out_hbm,
                    src_v, dst_v, rows_v, acc_sh, sem):
    c = lax.axis_index("c")
    s = lax.axis_index("s")
    w = s * NC + c

    pltpu.sync_copy(zero_hbm.at[pl.ds(s * ZERO_ROWS_PER_TILE, ZERO_ROWS_PER_TILE)],
                    acc_sh.at[pl.ds(s * ZERO_ROWS_PER_TILE, ZERO_ROWS_PER_TILE)])
    base = w * CHUNKS_PER_TILE
    pltpu.sync_copy(src_hbm.at[pl.ds(base, CHUNKS_PER_TILE)], src_v)
    pltpu.sync_copy(dst_hbm.at[pl.ds(base, CHUNKS_PER_TILE)], dst_v)
    plsc.subcore_barrier()

    def body(j, carry):
        pltpu.async_copy(x_hbm.at[src_v.at[j]], rows_v, sem).wait()
        pltpu.sync_copy(rows_v, acc_sh.at[dst_v.at[j]], add=True)
        return carry

    lax.fori_loop(0, CHUNKS_PER_TILE, body, 0)
    plsc.subcore_barrier()

    pltpu.sync_copy(acc_sh.at[pl.ds(s * OUT_ROWS_PER_TILE, OUT_ROWS_PER_TILE)],
                    out_hbm.at[c, pl.ds(s * OUT_ROWS_PER_TILE, OUT_ROWS_PER_TILE)])


def _finish_body(p_ref, w_ref, b_ref, o_ref):
    a = p_ref[0] + p_ref[1]
    o_ref[...] = jnp.dot(a, w_ref[...], preferred_element_type=jnp.float32) + b_ref[...]


def _tc_finish(partials, weight, bias):
    blk = 2000
    return pl.pallas_call(
        _finish_body,
        grid=(N_NODES // blk,),
        in_specs=[
            pl.BlockSpec((NC, blk, D), lambda i: (0, i, 0)),
            pl.BlockSpec((D, D), lambda i: (0, 0)),
            pl.BlockSpec((1, D), lambda i: (0, 0)),
        ],
        out_specs=pl.BlockSpec((blk, D), lambda i: (i, 0)),
        out_shape=jax.ShapeDtypeStruct((N_NODES, D), jnp.float32),
    )(partials, weight, bias.reshape(1, D))


def kernel(x, edge_index, weight, bias):
    dst = edge_index[0]
    src = edge_index[1]
    pad = E_PAD - N_EDGES
    src_p = jnp.concatenate([src, jnp.zeros((pad,), jnp.int32)]).reshape(N_CHUNKS_PAD, CHUNK)
    dst_p = jnp.concatenate([dst, jnp.full((pad,), N_NODES, jnp.int32)]).reshape(N_CHUNKS_PAD, CHUNK)
    zeros = jnp.zeros((ACC_ROWS, D), jnp.float32)
    partials = _sc_segment_sum(x, src_p, dst_p, zeros)
    return _tc_finish(partials, weight, bias)


# speedup vs baseline: 3.0065x; 3.0065x over previous
import functools

import jax
import jax.numpy as jnp
from jax import lax
from jax.experimental import pallas as pl
from jax.experimental.pallas import tpu as pltpu
from jax.experimental.pallas import tpu_sc as plsc

N_NODES = 10000
N_EDGES = 320000
D = 128

NC = 2
NS = 16
NW = NC * NS

CHUNK = 128
N_CHUNKS = N_EDGES // CHUNK
CHUNKS_PER_TILE = 80
N_CHUNKS_PAD = CHUNKS_PER_TILE * NW
E_PAD = N_CHUNKS_PAD * CHUNK

ACC_ROWS = 10112
ROWS_PER_TILE = ACC_ROWS // NS


@functools.partial(
    pl.kernel,
    out_type=jax.ShapeDtypeStruct((NC, ACC_ROWS, D), jnp.float32),
    mesh=plsc.VectorSubcoreMesh(core_axis_name="c", subcore_axis_name="s"),
    scratch_types=[
        pltpu.VMEM((CHUNKS_PER_TILE, CHUNK), jnp.int32),
        pltpu.VMEM((CHUNKS_PER_TILE, CHUNK), jnp.int32),
        pltpu.VMEM((CHUNK, D), jnp.float32),
        pltpu.VMEM_SHARED((ACC_ROWS, D), jnp.float32),
        pltpu.SemaphoreType.DMA,
    ],
)
def _sc_segment_sum(x_hbm, src_hbm, dst_hbm, zero_hbm, out_hbm,
                    src_v, dst_v, rows_v, acc_sh, sem):
    c = lax.axis_index("c")
    s = lax.axis_index("s")
    w = s * NC + c

    pltpu.sync_copy(zero_hbm.at[pl.ds(s * ROWS_PER_TILE, ROWS_PER_TILE)],
                    acc_sh.at[pl.ds(s * ROWS_PER_TILE, ROWS_PER_TILE)])
    base = w * CHUNKS_PER_TILE
    pltpu.sync_copy(src_hbm.at[pl.ds(base, CHUNKS_PER_TILE)], src_v)
    pltpu.sync_copy(dst_hbm.at[pl.ds(base, CHUNKS_PER_TILE)], dst_v)
    plsc.subcore_barrier()

    def body(j, carry):
        pltpu.async_copy(x_hbm.at[src_v.at[j]], rows_v, sem).wait()
        pltpu.sync_copy(rows_v, acc_sh.at[dst_v.at[j]], add=True)
        return carry

    lax.fori_loop(0, CHUNKS_PER_TILE, body, 0)
    plsc.subcore_barrier()

    pltpu.sync_copy(acc_sh.at[pl.ds(s * ROWS_PER_TILE, ROWS_PER_TILE)],
                    out_hbm.at[c, pl.ds(s * ROWS_PER_TILE, ROWS_PER_TILE)])


def _finish_body(p_ref, w_ref, b_ref, o_ref):
    a = p_ref[0] + p_ref[1]
    o_ref[...] = jnp.dot(a, w_ref[...], preferred_element_type=jnp.float32) + b_ref[...]


def _tc_finish(partials, weight, bias):
    blk = 2000
    return pl.pallas_call(
        _finish_body,
        grid=(N_NODES // blk,),
        in_specs=[
            pl.BlockSpec((NC, blk, D), lambda i: (0, i, 0)),
            pl.BlockSpec((D, D), lambda i: (0, 0)),
            pl.BlockSpec((1, D), lambda i: (0, 0)),
        ],
        out_specs=pl.BlockSpec((blk, D), lambda i: (i, 0)),
        out_shape=jax.ShapeDtypeStruct((N_NODES, D), jnp.float32),
    )(partials, weight, bias.reshape(1, D))


def kernel(x, edge_index, weight, bias):
    dst = edge_index[0]
    src = edge_index[1]
    pad = E_PAD - N_EDGES
    src_p = jnp.concatenate([src, jnp.zeros((pad,), jnp.int32)]).reshape(N_CHUNKS_PAD, CHUNK)
    dst_p = jnp.concatenate([dst, jnp.full((pad,), N_NODES, jnp.int32)]).reshape(N_CHUNKS_PAD, CHUNK)
    zeros = jnp.zeros((ACC_ROWS, D), jnp.float32)
    partials = _sc_segment_sum(x, src_p, dst_p, zeros)
    return _tc_finish(partials, weight, bias)


# speedup vs baseline: 3.4242x; 1.1389x over previous
import functools

import jax
import jax.numpy as jnp
from jax import lax
from jax.experimental import pallas as pl
from jax.experimental.pallas import tpu as pltpu
from jax.experimental.pallas import tpu_sc as plsc

N_NODES = 10000
N_EDGES = 320000
D = 128

NC = 2
NS = 16
NW = NC * NS
COLS = D // NW
L = 16

CH = 8000
N_CH = N_EDGES // CH
GROUPS = CH // L
TILE_W = COLS * N_NODES


@functools.partial(
    pl.kernel,
    out_type=jax.ShapeDtypeStruct((NW * TILE_W,), jnp.float32),
    mesh=plsc.VectorSubcoreMesh(core_axis_name="c", subcore_axis_name="s"),
    compiler_params=pltpu.CompilerParams(needs_layout_passes=False),
    scratch_types=[
        pltpu.VMEM((TILE_W,), jnp.float32),
        pltpu.VMEM((TILE_W,), jnp.float32),
        pltpu.VMEM((CH,), jnp.int32),
        pltpu.VMEM((CH,), jnp.int32),
        pltpu.VMEM((CH,), jnp.int32),
        pltpu.VMEM((CH,), jnp.int32),
        pltpu.SemaphoreType.DMA,
        pltpu.SemaphoreType.DMA,
        pltpu.SemaphoreType.DMA,
    ],
)
def _sc_segment_sum_t(xt_hbm, src_hbm, dst_hbm, out_hbm,
                      xcols, acc, src_a, dst_a, src_b, dst_b,
                      sem_x, sem_a, sem_b):
    c = lax.axis_index("c")
    s = lax.axis_index("s")
    t = s * NC + c

    xcp = pltpu.async_copy(xt_hbm.at[pl.ds(t * TILE_W, TILE_W)], xcols, sem_x)
    zero = jnp.zeros((L,), jnp.float32)

    def zbody(i, carry):
        acc[pl.ds(i * L, L)] = zero
        return carry

    lax.fori_loop(0, TILE_W // L, zbody, 0)
    xcp.wait()

    def fire(k, sbuf, dbuf, sem):
        pltpu.async_copy(src_hbm.at[pl.ds(k * CH, CH)], sbuf, sem)
        pltpu.async_copy(dst_hbm.at[pl.ds(k * CH, CH)], dbuf, sem)

    def drain(k, sbuf, dbuf, sem):
        pltpu.make_async_copy(src_hbm.at[pl.ds(k * CH, CH)], sbuf, sem).wait()
        pltpu.make_async_copy(dst_hbm.at[pl.ds(k * CH, CH)], dbuf, sem).wait()

    def process(sbuf, dbuf):
        def gbody(g, carry):
            s_vec = sbuf[pl.ds(g * L, L)]
            d_vec = dbuf[pl.ds(g * L, L)]
            for cc in range(COLS):
                v = plsc.load_gather(xcols, [s_vec + (cc * N_NODES)])
                plsc.addupdate_scatter(acc, [d_vec + (cc * N_NODES)], v)
            return carry

        lax.fori_loop(0, GROUPS, gbody, 0)

    fire(0, src_a, dst_a, sem_a)
    def mbody(m, carry):
        fire(2 * m + 1, src_b, dst_b, sem_b)
        drain(2 * m, src_a, dst_a, sem_a)
        process(src_a, dst_a)
        ka = jnp.minimum(2 * m + 2, N_CH - 1)
        fire(ka, src_a, dst_a, sem_a)
        drain(2 * m + 1, src_b, dst_b, sem_b)
        process(src_b, dst_b)
        return carry

    lax.fori_loop(0, N_CH // 2, mbody, 0)
    drain(N_CH - 1, src_a, dst_a, sem_a)

    pltpu.sync_copy(acc, out_hbm.at[pl.ds(t * TILE_W, TILE_W)])


def _finish_body(at_ref, w_ref, b_ref, o_ref):
    o_ref[...] = lax.dot_general(
        at_ref[...], w_ref[...],
        dimension_numbers=(((0,), (0,)), ((), ())),
        preferred_element_type=jnp.float32,
    ) + b_ref[...]


def _tc_finish(agg_t, weight, bias):
    return pl.pallas_call(
        _finish_body,
        out_shape=jax.ShapeDtypeStruct((N_NODES, D), jnp.float32),
    )(agg_t, weight, bias.reshape(1, D))


def kernel(x, edge_index, weight, bias):
    dst = edge_index[0]
    src = edge_index[1]
    xt_flat = x.T.reshape(NW * TILE_W)
    agg_flat = _sc_segment_sum_t(xt_flat, src, dst)
    agg_t = agg_flat.reshape(D, N_NODES)
    return _tc_finish(agg_t, weight, bias)
